# baseline (speedup 1.0000x reference)
import functools

import jax
import jax.numpy as jnp
from jax.experimental import pallas as pl
from jax.experimental.pallas import tpu as pltpu


def _router_kernel(ts_ref, wr_ref, br_ref, probs_ref, ti_ref, tv_ref, aux_ref):
    pooled = jnp.mean(ts_ref[...], axis=1)
    logits = jnp.dot(pooled, wr_ref[...],
                     preferred_element_type=jnp.float32) + br_ref[...]
    probs = jax.nn.softmax(logits, axis=-1)
    bsz, num_e = probs.shape
    iota = jax.lax.broadcasted_iota(jnp.int32, (bsz, num_e), 1)
    v1 = jnp.max(probs, axis=-1, keepdims=True)
    i1 = jnp.min(jnp.where(probs == v1, iota, num_e), axis=-1, keepdims=True)
    masked = jnp.where(iota == i1, -1.0, probs)
    v2 = jnp.max(masked, axis=-1, keepdims=True)
    i2 = jnp.min(jnp.where(masked == v2, iota, num_e), axis=-1, keepdims=True)
    denom = v1 + v2
    tv = jnp.concatenate([v1 / denom, v2 / denom], axis=1)
    ti = jnp.concatenate([i1, i2], axis=1)
    probs_ref[...] = probs
    ti_ref[...] = ti
    tv_ref[...] = tv
    me = jnp.mean(probs, axis=0)
    aux_ref[...] = (jnp.sum(me * me) * num_e).reshape(1, 1)


def _moe_kernel(nf, idx_ref, wv_ref, x_ref, w1_ref, b1_ref, w2_ref, b2_ref,
                g_ref, bt_ref, out_ref):
    i_bk = pl.program_id(0)
    i_f = pl.program_id(1)
    w = wv_ref[i_bk]

    @pl.when(jnp.logical_and(i_bk % 2 == 0, i_f == 0))
    def _init():
        out_ref[0] = jnp.zeros_like(out_ref[0])

    t = jnp.dot(x_ref[0], w1_ref[0].astype(jnp.bfloat16),
                preferred_element_type=jnp.float32).astype(jnp.bfloat16)
    t = t + b1_ref[0].astype(jnp.bfloat16)
    c1 = jnp.bfloat16(0.7978845608028654)
    c2 = jnp.bfloat16(0.7978845608028654 * 0.044715)
    t2 = t * t
    u = t * (c1 + c2 * t2)
    th = jnp.tanh(u)
    half = jnp.bfloat16(0.5)
    h = t * (half + half * th)
    part = jnp.dot(h, (w * w2_ref[0]).astype(jnp.bfloat16),
                   preferred_element_type=jnp.float32)

    @pl.when(i_f == 0)
    def _bias():
        out_ref[0] += w * b2_ref[0]

    out_ref[0] += part

    @pl.when(jnp.logical_and(i_bk % 2 == 1, i_f == nf - 1))
    def _epilogue():
        mixed = out_ref[0]
        mu = jnp.mean(mixed, axis=-1, keepdims=True)
        var = jnp.mean((mixed - mu) ** 2, axis=-1, keepdims=True)
        normed = (mixed - mu) * jax.lax.rsqrt(var + 1e-5)
        out_ref[0] = (x_ref[0].astype(jnp.float32)
                      + normed * g_ref[...] + bt_ref[...])


def kernel(x, text_state, W1, b1, W2, b2, Wr, br, gamma, beta):
    b, s, d = x.shape
    E, _, ff = W1.shape
    k = 2

    probs, ti, tv, aux = pl.pallas_call(
        _router_kernel,
        out_shape=(
            jax.ShapeDtypeStruct((b, E), jnp.float32),
            jax.ShapeDtypeStruct((b, k), jnp.int32),
            jax.ShapeDtypeStruct((b, k), jnp.float32),
            jax.ShapeDtypeStruct((1, 1), jnp.float32),
        ),
    )(text_state, Wr, br.reshape(1, E))

    ft = 1024
    nf = ff // ft
    b1r = b1.reshape(E, 1, ff)
    b2r = b2.reshape(E, 1, d)

    grid_spec = pltpu.PrefetchScalarGridSpec(
        num_scalar_prefetch=2,
        grid=(k * b, nf),
        in_specs=[
            pl.BlockSpec((1, s, d), lambda i, j, idx, wv: (i // 2, 0, 0),
                         pipeline_mode=pl.Buffered(buffer_count=1)),
            pl.BlockSpec((1, d, ft), lambda i, j, idx, wv: (idx[i], 0, j)),
            pl.BlockSpec((1, 1, ft), lambda i, j, idx, wv: (idx[i], 0, j)),
            pl.BlockSpec((1, ft, d), lambda i, j, idx, wv: (idx[i], j, 0)),
            pl.BlockSpec((1, 1, d), lambda i, j, idx, wv: (idx[i], 0, 0)),
            pl.BlockSpec((1, d), lambda i, j, idx, wv: (0, 0)),
            pl.BlockSpec((1, d), lambda i, j, idx, wv: (0, 0)),
        ],
        out_specs=pl.BlockSpec((1, s, d), lambda i, j, idx, wv: (i // 2, 0, 0),
                               pipeline_mode=pl.Buffered(buffer_count=1)),
    )

    out = pl.pallas_call(
        functools.partial(_moe_kernel, nf),
        grid_spec=grid_spec,
        out_shape=jax.ShapeDtypeStruct((b, s, d), jnp.float32),
        compiler_params=pltpu.CompilerParams(
            vmem_limit_bytes=63 * 1024 * 1024),
    )(ti.reshape(k * b), tv.reshape(k * b), x.astype(jnp.bfloat16),
      W1, b1r, W2, b2r, gamma.reshape(1, d), beta.reshape(1, d))

    return out, probs, aux.reshape(())


# speedup vs baseline: 1.0464x; 1.0464x over previous
import functools

import jax
import jax.numpy as jnp
from jax.experimental import pallas as pl
from jax.experimental.pallas import tpu as pltpu


def _router_kernel(ts_ref, wr_ref, br_ref, probs_ref, ti_ref, tv_ref, aux_ref):
    pooled = jnp.mean(ts_ref[...], axis=1)
    logits = jnp.dot(pooled, wr_ref[...],
                     preferred_element_type=jnp.float32) + br_ref[...]
    probs = jax.nn.softmax(logits, axis=-1)
    bsz, num_e = probs.shape
    iota = jax.lax.broadcasted_iota(jnp.int32, (bsz, num_e), 1)
    v1 = jnp.max(probs, axis=-1, keepdims=True)
    i1 = jnp.min(jnp.where(probs == v1, iota, num_e), axis=-1, keepdims=True)
    masked = jnp.where(iota == i1, -1.0, probs)
    v2 = jnp.max(masked, axis=-1, keepdims=True)
    i2 = jnp.min(jnp.where(masked == v2, iota, num_e), axis=-1, keepdims=True)
    denom = v1 + v2
    tv = jnp.concatenate([v1 / denom, v2 / denom], axis=1)
    ti = jnp.concatenate([i1, i2], axis=1)
    probs_ref[...] = probs
    ti_ref[...] = ti
    tv_ref[...] = tv
    me = jnp.mean(probs, axis=0)
    aux_ref[...] = (jnp.sum(me * me) * num_e).reshape(1, 1)


def _moe_kernel(nf, idx_ref, wv_ref, x_ref, w1_ref, b1_ref, w2_ref, b2_ref,
                g_ref, bt_ref, out_ref):
    i_bk = pl.program_id(0)
    i_f = pl.program_id(1)
    w = wv_ref[i_bk]

    @pl.when(jnp.logical_and(i_bk % 2 == 0, i_f == 0))
    def _init():
        out_ref[0] = jnp.zeros_like(out_ref[0])

    t = jnp.dot(x_ref[0], w1_ref[0],
                preferred_element_type=jnp.float32).astype(jnp.bfloat16)
    t = t + b1_ref[0].astype(jnp.bfloat16)
    c1 = jnp.bfloat16(0.7978845608028654)
    c2 = jnp.bfloat16(0.7978845608028654 * 0.044715)
    t2 = t * t
    u = t * (c1 + c2 * t2)
    th = jnp.tanh(u)
    half = jnp.bfloat16(0.5)
    h = t * (half + half * th)
    part = jnp.dot(h, (w * w2_ref[0]).astype(jnp.bfloat16),
                   preferred_element_type=jnp.float32)

    @pl.when(i_f == 0)
    def _bias():
        out_ref[0] += w * b2_ref[0]

    out_ref[0] += part

    @pl.when(jnp.logical_and(i_bk % 2 == 1, i_f == nf - 1))
    def _epilogue():
        mixed = out_ref[0]
        mu = jnp.mean(mixed, axis=-1, keepdims=True)
        var = jnp.mean((mixed - mu) ** 2, axis=-1, keepdims=True)
        normed = (mixed - mu) * jax.lax.rsqrt(var + 1e-5)
        out_ref[0] = x_ref[0] + normed * g_ref[...] + bt_ref[...]


def kernel(x, text_state, W1, b1, W2, b2, Wr, br, gamma, beta):
    b, s, d = x.shape
    E, _, ff = W1.shape
    k = 2

    probs, ti, tv, aux = pl.pallas_call(
        _router_kernel,
        out_shape=(
            jax.ShapeDtypeStruct((b, E), jnp.float32),
            jax.ShapeDtypeStruct((b, k), jnp.int32),
            jax.ShapeDtypeStruct((b, k), jnp.float32),
            jax.ShapeDtypeStruct((1, 1), jnp.float32),
        ),
    )(text_state, Wr, br.reshape(1, E))

    ft = 1024
    nf = ff // ft
    b1r = b1.reshape(E, 1, ff)
    b2r = b2.reshape(E, 1, d)

    grid_spec = pltpu.PrefetchScalarGridSpec(
        num_scalar_prefetch=2,
        grid=(k * b, nf),
        in_specs=[
            pl.BlockSpec((1, s, d), lambda i, j, idx, wv: (i // 2, 0, 0),
                         pipeline_mode=pl.Buffered(buffer_count=1)),
            pl.BlockSpec((1, d, ft), lambda i, j, idx, wv: (idx[i], 0, j)),
            pl.BlockSpec((1, 1, ft), lambda i, j, idx, wv: (idx[i], 0, j)),
            pl.BlockSpec((1, ft, d), lambda i, j, idx, wv: (idx[i], j, 0)),
            pl.BlockSpec((1, 1, d), lambda i, j, idx, wv: (idx[i], 0, 0)),
            pl.BlockSpec((1, d), lambda i, j, idx, wv: (0, 0)),
            pl.BlockSpec((1, d), lambda i, j, idx, wv: (0, 0)),
        ],
        out_specs=pl.BlockSpec((1, s, d), lambda i, j, idx, wv: (i // 2, 0, 0),
                               pipeline_mode=pl.Buffered(buffer_count=1)),
    )

    out = pl.pallas_call(
        functools.partial(_moe_kernel, nf),
        grid_spec=grid_spec,
        out_shape=jax.ShapeDtypeStruct((b, s, d), jnp.float32),
        compiler_params=pltpu.CompilerParams(
            vmem_limit_bytes=63 * 1024 * 1024),
    )(ti.reshape(k * b), tv.reshape(k * b), x, W1, b1r, W2, b2r,
      gamma.reshape(1, d), beta.reshape(1, d))

    return out, probs, aux.reshape(())
